# baseline (speedup 1.0000x reference)
import functools

import jax
import jax.numpy as jnp
from jax import lax
from jax.experimental import pallas as pl
from jax.experimental.pallas import tpu as pltpu
from jax.experimental.pallas import tpu_sc as plsc

N = 10000
E = 160000
D = 256
H = 4
F = 64
NEG_SLOPE = 0.2

BN = 1000
CW = 136
CH = 80
EPT = E // 16
NCHUNK = EPT // CH
RPT = 624
ZR = 52


def _tc1_body(x_ref, w_ref, b_ref, feat_ref, elr_ref):
    xb = x_ref[...]
    wb = w_ref[0]
    fb = lax.dot_general(xb, wb, (((1,), (1,)), ((), ())),
                         preferred_element_type=jnp.float32)
    ee = jnp.dot(fb, b_ref[0, 0], preferred_element_type=jnp.float32)
    elr_ref[0, 0] = ee
    feat_ref[0, 0] = jnp.concatenate(
        [fb, ee[:, 0:2], jnp.zeros((BN, CW - 130), jnp.float32)], axis=1)


def _stage1(x, Wstack, Ball):
    return pl.pallas_call(
        _tc1_body,
        grid=(2, 2, N // BN),
        in_specs=[
            pl.BlockSpec((BN, D), lambda r, c, n: (n, 0)),
            pl.BlockSpec((1, 128, D), lambda r, c, n: (r, c, 0)),
            pl.BlockSpec((1, 1, 128, 16), lambda r, c, n: (r, c, 0, 0)),
        ],
        out_specs=[
            pl.BlockSpec((1, 1, BN, CW), lambda r, c, n: (r, c, n, 0)),
            pl.BlockSpec((1, 1, BN, 16), lambda r, c, n: (r, c, n, 0)),
        ],
        out_shape=[
            jax.ShapeDtypeStruct((2, 2, N, CW), jnp.float32),
            jax.ShapeDtypeStruct((2, 2, N, 16), jnp.float32),
        ],
    )(x, Wstack, Ball)


def _sc_body(feat_hbm, elr_hbm, ei_hbm, out_hbm,
             gbuf0, gbuf1, edb0, edb1,
             srcb0, srcb1, dstb0, dstb1, fidx0, fidx1, didx0, didx1,
             dsc0, dsc1, zbuf2, acc,
             sid0, sid1, sg0, sg1, ss0, ss1):
    c = lax.axis_index("c")
    s = lax.axis_index("s")
    GB = (gbuf0, gbuf1)
    ED = (edb0, edb1)
    SR = (srcb0, srcb1)
    DS = (dstb0, dstb1)
    FI = (fidx0, fidx1)
    DI = (didx0, didx1)
    DC = (dsc0, dsc1)
    SID = (sid0, sid1)
    SG = (sg0, sg1)
    SS = (ss0, ss1)

    def _z2(i, _):
        for v in range(CW // 8 // 2):
            zbuf2[i, pl.ds(v * 16, 16)] = jnp.zeros((16,), jnp.float32)
        zbuf2[i, pl.ds(CW - 16, 16)] = jnp.zeros((16,), jnp.float32)
        return _
    lax.fori_loop(0, ZR, _z2, None)

    for r in (0, 1):
        row_base = s * RPT
        for q in range(RPT // ZR):
            pltpu.async_copy(zbuf2, acc.at[pl.ds(row_base + q * ZR, ZR)], sg0)
        for q in range(RPT // ZR):
            pltpu.make_async_copy(
                zbuf2, acc.at[pl.ds(row_base + q * ZR, ZR)], sg0).wait()

        @pl.when(s == 15)
        def _ztail():
            pltpu.sync_copy(zbuf2.at[pl.ds(0, 16)], acc.at[pl.ds(N - 16, 16)])

        plsc.subcore_barrier()

        feat_base = (2 * r + c) * N
        srcs_hbm = ei_hbm.at[r, 0]
        dsts_hbm = ei_hbm.at[r, 1]

        def issue_ids(k, p):
            base = s * EPT + k * CH
            pltpu.async_copy(srcs_hbm.at[pl.ds(base, CH)], SR[p], SID[p])
            pltpu.async_copy(dsts_hbm.at[pl.ds(base, CH)], DS[p], SID[p])

        def wait_ids(p):
            pltpu.make_async_copy(srcs_hbm.at[pl.ds(0, CH)], SR[p], SID[p]).wait()
            pltpu.make_async_copy(dsts_hbm.at[pl.ds(0, CH)], DS[p], SID[p]).wait()

        def build_and_gather(p):
            for g in range(CH // 16):
                sl = pl.ds(g * 16, 16)
                sv = SR[p][sl]
                dv = DS[p][sl]
                FI[p][sl] = sv + feat_base
                DI[p][sl] = dv + feat_base
                DC[p][sl] = dv
            pltpu.async_copy(feat_hbm.at[FI[p]], GB[p], SG[p])
            pltpu.async_copy(elr_hbm.at[DI[p]], ED[p], SG[p])

        def wait_gather(p):
            pltpu.make_async_copy(feat_hbm.at[FI[p]], GB[p], SG[p]).wait()
            pltpu.make_async_copy(elr_hbm.at[DI[p]], ED[p], SG[p]).wait()

        def process(p):
            def _grp(g, _):
                i16 = lax.iota(jnp.int32, 16) + g * 16
                exv = []
                for j in range(2):
                    el = plsc.load_gather(
                        GB[p], [i16, jnp.full((16,), 128 + j, jnp.int32)])
                    er = plsc.load_gather(
                        ED[p], [i16, jnp.full((16,), 2 + j, jnp.int32)])
                    e = el + er
                    e = jnp.where(e >= 0, e, e * NEG_SLOPE)
                    ex = jnp.exp(e)
                    plsc.store_scatter(
                        GB[p], [i16, jnp.full((16,), 128 + j, jnp.int32)], ex)
                    exv.append(ex)
                for l in range(16):
                    i = g * 16 + l
                    w0 = exv[0][l]
                    w1 = exv[1][l]
                    for v in range(4):
                        GB[p][i, pl.ds(v * 16, 16)] = (
                            GB[p][i, pl.ds(v * 16, 16)] * w0)
                        GB[p][i, pl.ds(64 + v * 16, 16)] = (
                            GB[p][i, pl.ds(64 + v * 16, 16)] * w1)
                return _
            lax.fori_loop(0, CH // 16, _grp, None)

        def issue_scatter(p):
            pltpu.async_copy(GB[p], acc.at[DC[p]], SS[p], add=True)

        def drain_scatter(p):
            pltpu.make_async_copy(GB[p], acc.at[DC[p]], SS[p]).wait()

        issue_ids(0, 0)
        wait_ids(0)
        build_and_gather(0)
        issue_ids(1, 1)
        wait_ids(1)
        build_and_gather(1)
        wait_gather(0)
        process(0)
        issue_scatter(0)
        issue_ids(2, 0)

        def _main(m, _):
            k = 2 * m
            issue_ids(k + 3, 1)
            drain_scatter(0)
            wait_ids(0)
            build_and_gather(0)
            wait_gather(1)
            process(1)
            issue_scatter(1)
            issue_ids(k + 4, 0)
            drain_scatter(1)
            wait_ids(1)
            build_and_gather(1)
            wait_gather(0)
            process(0)
            issue_scatter(0)
            return _
        lax.fori_loop(0, (NCHUNK - 3) // 2, _main, None)

        drain_scatter(0)
        wait_ids(0)
        build_and_gather(0)
        wait_gather(1)
        process(1)
        issue_scatter(1)
        drain_scatter(1)
        wait_gather(0)
        process(0)
        issue_scatter(0)
        drain_scatter(0)

        plsc.subcore_barrier()

        r0 = s * RPT
        pltpu.sync_copy(acc.at[pl.ds(r0, RPT)],
                        out_hbm.at[r, pl.ds(c * N + r0, RPT)])

        @pl.when(s == 15)
        def _wtail():
            pltpu.sync_copy(acc.at[pl.ds(N - 16, 16)],
                            out_hbm.at[r, pl.ds(c * N + N - 16, 16)])

        plsc.subcore_barrier()


def _stage2(feat_cat, elr, ei_all):
    mesh = plsc.VectorSubcoreMesh(core_axis_name="c", subcore_axis_name="s")
    fn = pl.kernel(
        _sc_body,
        out_type=jax.ShapeDtypeStruct((2, 2 * N, CW), jnp.float32),
        mesh=mesh,
        compiler_params=pltpu.CompilerParams(use_tc_tiling_on_sc=False,
                                             needs_layout_passes=False),
        scratch_types=(
            [pltpu.VMEM((CH, CW), jnp.float32)] * 2 +
            [pltpu.VMEM((CH, 16), jnp.float32)] * 2 +
            [pltpu.VMEM((CH,), jnp.int32)] * 10 +
            [pltpu.VMEM((ZR, CW), jnp.float32),
             pltpu.VMEM_SHARED((N, CW), jnp.float32)] +
            [pltpu.SemaphoreType.DMA] * 6
        ),
    )
    return fn(feat_cat, elr, ei_all)


def _tc2_body(h_ref, wm_ref, bm_ref, o_ref):
    hb = h_ref[...]
    parts = []
    for q in range(4):
        dn = hb[q][:, 128:130]
        dn = jnp.where(dn == 0.0, 1.0, dn)
        inv = 1.0 / dn
        i0 = jnp.broadcast_to(inv[:, 0:1], (BN, 64))
        i1 = jnp.broadcast_to(inv[:, 1:2], (BN, 64))
        parts.append(hb[q][:, :128] * jnp.concatenate([i0, i1], axis=1))
    cat = jnp.concatenate(parts, axis=1)
    o_ref[...] = lax.dot_general(cat, wm_ref[...], (((1,), (1,)), ((), ())),
                                 preferred_element_type=jnp.float32) + bm_ref[...]


def _stage3(hcat, Wm, bm):
    return pl.pallas_call(
        _tc2_body,
        grid=(N // BN,),
        in_specs=[
            pl.BlockSpec((4, BN, CW), lambda n: (0, n, 0)),
            pl.BlockSpec((F, 2 * H * F), lambda n: (0, 0)),
            pl.BlockSpec((1, F), lambda n: (0, 0)),
        ],
        out_specs=pl.BlockSpec((BN, F), lambda n: (n, 0)),
        out_shape=jax.ShapeDtypeStruct((N, F), jnp.float32),
    )(hcat, Wm, bm)


def _build_b(attn_l, attn_r):
    z = jnp.zeros((64,), jnp.float32)
    zcol = jnp.zeros((128,), jnp.float32)
    per_c = []
    for c in range(2):
        cols = [jnp.concatenate([attn_l[0, 2 * c], z]),
                jnp.concatenate([z, attn_l[0, 2 * c + 1]]),
                jnp.concatenate([attn_r[0, 2 * c], z]),
                jnp.concatenate([z, attn_r[0, 2 * c + 1]])] + [zcol] * 12
        per_c.append(jnp.stack(cols, axis=1))
    return jnp.stack(per_c)


def kernel(x, edge_index_rel0, edge_index_rel1, W1, attn_l1, attn_r1,
           W2, attn_l2, attn_r2, Wm, bm):
    Wstack = jnp.stack([W1, W2])
    Ball = jnp.stack([_build_b(attn_l1, attn_r1),
                      _build_b(attn_l2, attn_r2)])
    ei_all = jnp.stack([edge_index_rel0, edge_index_rel1])

    feat, elr = _stage1(x, Wstack, Ball)
    feat_cat = feat.reshape(4 * N, CW)
    elr_cat = elr.reshape(4 * N, 16)

    out_raw = _stage2(feat_cat, elr_cat, ei_all)
    hcat = out_raw.reshape(4, N, CW)

    return _stage3(hcat, Wm, bm.reshape(1, F))


# speedup vs baseline: 1.0239x; 1.0239x over previous
import functools

import jax
import jax.numpy as jnp
from jax import lax
from jax.experimental import pallas as pl
from jax.experimental.pallas import tpu as pltpu
from jax.experimental.pallas import tpu_sc as plsc

N = 10000
E = 160000
D = 256
H = 4
F = 64
NEG_SLOPE = 0.2

BN = 1000
CW = 144
CH = 80
EPT = E // 16
NCHUNK = EPT // CH
RPT = 624
ZR = 52


def _tc1_body(x_ref, w_ref, b_ref, feat_ref, elr_ref):
    xb = x_ref[...]
    wb = w_ref[0]
    fb = lax.dot_general(xb, wb, (((1,), (1,)), ((), ())),
                         preferred_element_type=jnp.float32)
    ee = jnp.dot(fb, b_ref[0, 0], preferred_element_type=jnp.float32)
    elr_ref[0, 0] = ee
    feat_ref[0, 0] = jnp.concatenate(
        [fb, ee[:, 0:2], jnp.zeros((BN, CW - 130), jnp.float32)], axis=1)


def _stage1(x, Wstack, Ball):
    return pl.pallas_call(
        _tc1_body,
        grid=(2, 2, N // BN),
        in_specs=[
            pl.BlockSpec((BN, D), lambda r, c, n: (n, 0)),
            pl.BlockSpec((1, 128, D), lambda r, c, n: (r, c, 0)),
            pl.BlockSpec((1, 1, 128, 16), lambda r, c, n: (r, c, 0, 0)),
        ],
        out_specs=[
            pl.BlockSpec((1, 1, BN, CW), lambda r, c, n: (r, c, n, 0)),
            pl.BlockSpec((1, 1, BN, 16), lambda r, c, n: (r, c, n, 0)),
        ],
        out_shape=[
            jax.ShapeDtypeStruct((2, 2, N, CW), jnp.float32),
            jax.ShapeDtypeStruct((2, 2, N, 16), jnp.float32),
        ],
    )(x, Wstack, Ball)


def _sc_body(feat_hbm, elr_hbm, ei_hbm, out_hbm,
             gbuf0, gbuf1, edb0, edb1,
             srcb0, srcb1, dstb0, dstb1, fidx0, fidx1, didx0, didx1,
             dsc0, dsc1, zbuf2, acc,
             sid0, sid1, sg0, sg1, ss0, ss1):
    c = lax.axis_index("c")
    s = lax.axis_index("s")
    GB = (gbuf0, gbuf1)
    ED = (edb0, edb1)
    SR = (srcb0, srcb1)
    DS = (dstb0, dstb1)
    FI = (fidx0, fidx1)
    DI = (didx0, didx1)
    DC = (dsc0, dsc1)
    SID = (sid0, sid1)
    SG = (sg0, sg1)
    SS = (ss0, ss1)

    def _z2(i, _):
        for v in range(CW // 8 // 2):
            zbuf2[i, pl.ds(v * 16, 16)] = jnp.zeros((16,), jnp.float32)
        zbuf2[i, pl.ds(CW - 16, 16)] = jnp.zeros((16,), jnp.float32)
        return _
    lax.fori_loop(0, ZR, _z2, None)

    for r in (0, 1):
        row_base = s * RPT
        for q in range(RPT // ZR):
            pltpu.async_copy(zbuf2, acc.at[pl.ds(row_base + q * ZR, ZR)], sg0)
        for q in range(RPT // ZR):
            pltpu.make_async_copy(
                zbuf2, acc.at[pl.ds(row_base + q * ZR, ZR)], sg0).wait()

        @pl.when(s == 15)
        def _ztail():
            pltpu.sync_copy(zbuf2.at[pl.ds(0, 16)], acc.at[pl.ds(N - 16, 16)])

        plsc.subcore_barrier()

        feat_base = (2 * r + c) * N
        srcs_hbm = ei_hbm.at[r, 0]
        dsts_hbm = ei_hbm.at[r, 1]

        def issue_ids(k, p):
            base = s * EPT + k * CH
            pltpu.async_copy(srcs_hbm.at[pl.ds(base, CH)], SR[p], SID[p])
            pltpu.async_copy(dsts_hbm.at[pl.ds(base, CH)], DS[p], SID[p])

        def wait_ids(p):
            pltpu.make_async_copy(srcs_hbm.at[pl.ds(0, CH)], SR[p], SID[p]).wait()
            pltpu.make_async_copy(dsts_hbm.at[pl.ds(0, CH)], DS[p], SID[p]).wait()

        def build_and_gather(p):
            for g in range(CH // 16):
                sl = pl.ds(g * 16, 16)
                sv = SR[p][sl]
                dv = DS[p][sl]
                FI[p][sl] = sv + feat_base
                DI[p][sl] = dv + feat_base
                DC[p][sl] = dv
            pltpu.async_copy(feat_hbm.at[FI[p]], GB[p], SG[p])
            pltpu.async_copy(elr_hbm.at[DI[p]], ED[p], SG[p])

        def wait_gather(p):
            pltpu.make_async_copy(feat_hbm.at[FI[p]], GB[p], SG[p]).wait()
            pltpu.make_async_copy(elr_hbm.at[DI[p]], ED[p], SG[p]).wait()

        def process(p):
            def _grp(g, _):
                i16 = lax.iota(jnp.int32, 16) + g * 16
                exv = []
                for j in range(2):
                    el = plsc.load_gather(
                        GB[p], [i16, jnp.full((16,), 128 + j, jnp.int32)])
                    er = plsc.load_gather(
                        ED[p], [i16, jnp.full((16,), 2 + j, jnp.int32)])
                    e = el + er
                    e = jnp.where(e >= 0, e, e * NEG_SLOPE)
                    ex = jnp.exp(e)
                    plsc.store_scatter(
                        GB[p], [i16, jnp.full((16,), 128 + j, jnp.int32)], ex)
                    exv.append(ex)
                for l in range(16):
                    i = g * 16 + l
                    w0 = exv[0][l]
                    w1 = exv[1][l]
                    for v in range(4):
                        GB[p][i, pl.ds(v * 16, 16)] = (
                            GB[p][i, pl.ds(v * 16, 16)] * w0)
                        GB[p][i, pl.ds(64 + v * 16, 16)] = (
                            GB[p][i, pl.ds(64 + v * 16, 16)] * w1)
                return _
            lax.fori_loop(0, CH // 16, _grp, None)

        def issue_scatter(p):
            pltpu.async_copy(GB[p], acc.at[DC[p]], SS[p], add=True)

        def drain_scatter(p):
            pltpu.make_async_copy(GB[p], acc.at[DC[p]], SS[p]).wait()

        issue_ids(0, 0)
        wait_ids(0)
        build_and_gather(0)
        issue_ids(1, 1)
        wait_ids(1)
        build_and_gather(1)
        wait_gather(0)
        process(0)
        issue_scatter(0)
        issue_ids(2, 0)

        def _main(m, _):
            k = 2 * m
            issue_ids(k + 3, 1)
            drain_scatter(0)
            wait_ids(0)
            build_and_gather(0)
            wait_gather(1)
            process(1)
            issue_scatter(1)
            issue_ids(k + 4, 0)
            drain_scatter(1)
            wait_ids(1)
            build_and_gather(1)
            wait_gather(0)
            process(0)
            issue_scatter(0)
            return _
        lax.fori_loop(0, (NCHUNK - 3) // 2, _main, None)

        drain_scatter(0)
        wait_ids(0)
        build_and_gather(0)
        wait_gather(1)
        process(1)
        issue_scatter(1)
        drain_scatter(1)
        wait_gather(0)
        process(0)
        issue_scatter(0)
        drain_scatter(0)

        plsc.subcore_barrier()

        r0 = s * RPT
        pltpu.sync_copy(acc.at[pl.ds(r0, RPT)],
                        out_hbm.at[r, pl.ds(c * N + r0, RPT)])

        @pl.when(s == 15)
        def _wtail():
            pltpu.sync_copy(acc.at[pl.ds(N - 16, 16)],
                            out_hbm.at[r, pl.ds(c * N + N - 16, 16)])

        plsc.subcore_barrier()


def _stage2(feat_cat, elr, ei_all):
    mesh = plsc.VectorSubcoreMesh(core_axis_name="c", subcore_axis_name="s")
    fn = pl.kernel(
        _sc_body,
        out_type=jax.ShapeDtypeStruct((2, 2 * N, CW), jnp.float32),
        mesh=mesh,
        compiler_params=pltpu.CompilerParams(use_tc_tiling_on_sc=False,
                                             needs_layout_passes=False),
        scratch_types=(
            [pltpu.VMEM((CH, CW), jnp.float32)] * 2 +
            [pltpu.VMEM((CH, 16), jnp.float32)] * 2 +
            [pltpu.VMEM((CH,), jnp.int32)] * 10 +
            [pltpu.VMEM((ZR, CW), jnp.float32),
             pltpu.VMEM_SHARED((N, CW), jnp.float32)] +
            [pltpu.SemaphoreType.DMA] * 6
        ),
    )
    return fn(feat_cat, elr, ei_all)


def _tc2_body(h_ref, wm_ref, bm_ref, o_ref):
    hb = h_ref[...]
    parts = []
    for q in range(4):
        dn = hb[q][:, 128:130]
        dn = jnp.where(dn == 0.0, 1.0, dn)
        inv = 1.0 / dn
        i0 = jnp.broadcast_to(inv[:, 0:1], (BN, 64))
        i1 = jnp.broadcast_to(inv[:, 1:2], (BN, 64))
        parts.append(hb[q][:, :128] * jnp.concatenate([i0, i1], axis=1))
    cat = jnp.concatenate(parts, axis=1)
    o_ref[...] = lax.dot_general(cat, wm_ref[...], (((1,), (1,)), ((), ())),
                                 preferred_element_type=jnp.float32) + bm_ref[...]


def _stage3(hcat, Wm, bm):
    return pl.pallas_call(
        _tc2_body,
        grid=(N // BN,),
        in_specs=[
            pl.BlockSpec((4, BN, CW), lambda n: (0, n, 0)),
            pl.BlockSpec((F, 2 * H * F), lambda n: (0, 0)),
            pl.BlockSpec((1, F), lambda n: (0, 0)),
        ],
        out_specs=pl.BlockSpec((BN, F), lambda n: (n, 0)),
        out_shape=jax.ShapeDtypeStruct((N, F), jnp.float32),
    )(hcat, Wm, bm)


def _build_b(attn_l, attn_r):
    z = jnp.zeros((64,), jnp.float32)
    zcol = jnp.zeros((128,), jnp.float32)
    per_c = []
    for c in range(2):
        cols = [jnp.concatenate([attn_l[0, 2 * c], z]),
                jnp.concatenate([z, attn_l[0, 2 * c + 1]]),
                jnp.concatenate([attn_r[0, 2 * c], z]),
                jnp.concatenate([z, attn_r[0, 2 * c + 1]])] + [zcol] * 12
        per_c.append(jnp.stack(cols, axis=1))
    return jnp.stack(per_c)


def kernel(x, edge_index_rel0, edge_index_rel1, W1, attn_l1, attn_r1,
           W2, attn_l2, attn_r2, Wm, bm):
    Wstack = jnp.stack([W1, W2])
    Ball = jnp.stack([_build_b(attn_l1, attn_r1),
                      _build_b(attn_l2, attn_r2)])
    ei_all = jnp.stack([edge_index_rel0, edge_index_rel1])

    feat, elr = _stage1(x, Wstack, Ball)
    feat_cat = feat.reshape(4 * N, CW)
    elr_cat = elr.reshape(4 * N, 16)

    out_raw = _stage2(feat_cat, elr_cat, ei_all)
    hcat = out_raw.reshape(4, N, CW)

    return _stage3(hcat, Wm, bm.reshape(1, F))


# speedup vs baseline: 1.2145x; 1.1861x over previous
import functools

import jax
import jax.numpy as jnp
from jax import lax
from jax.experimental import pallas as pl
from jax.experimental.pallas import tpu as pltpu
from jax.experimental.pallas import tpu_sc as plsc

N = 10000
E = 160000
D = 256
H = 4
F = 64
NEG_SLOPE = 0.2

BN = 1000
CH = 80
EPT = E // 16
NCHUNK = EPT // CH
RPT = 624
ZR = 52


def _tc1_body(x_ref, w_ref, b_ref, feat_ref, elr_ref):
    xb = x_ref[...]
    wb = w_ref[0]
    fb = lax.dot_general(xb, wb, (((1,), (1,)), ((), ())),
                         preferred_element_type=jnp.float32)
    feat_ref[0, 0] = fb
    elr_ref[0, 0] = jnp.dot(fb, b_ref[0, 0], preferred_element_type=jnp.float32)


def _tc1_specs():
    return dict(
        in_specs=[
            pl.BlockSpec((BN, D), lambda n, r, c: (n, 0)),
            pl.BlockSpec((1, 128, D), lambda n, r, c: (r, c, 0)),
            pl.BlockSpec((1, 1, 128, 16), lambda n, r, c: (r, c, 0, 0)),
        ],
        out_specs=[
            pl.BlockSpec((1, 1, BN, 128), lambda n, r, c: (r, c, n, 0)),
            pl.BlockSpec((1, 1, BN, 16), lambda n, r, c: (r, c, n, 0)),
        ],
        out_shape=[
            jax.ShapeDtypeStruct((2, 2, N, 128), jnp.float32),
            jax.ShapeDtypeStruct((2, 2, N, 16), jnp.float32),
        ],
    )


def _stage1(x, Wstack, Ball):
    return pl.pallas_call(
        _tc1_body, grid=(N // BN, 2, 2), **_tc1_specs(),
    )(x, Wstack, Ball)


def _sc_body(feat_hbm, elr_hbm, ei_hbm, out_hbm, den_hbm,
             gbuf0, gbuf1, esb0, esb1, edb0, edb1,
             srcb0, srcb1, dstb0, dstb1, fidx0, fidx1, didx0, didx1,
             dsc0, dsc1, exb0, exb1, zbuf2, zbufn, acc, den0, den1,
             sid0, sid1, sg0, sg1, ss0, ss1):
    c = lax.axis_index("c")
    s = lax.axis_index("s")
    GB = (gbuf0, gbuf1)
    ES = (esb0, esb1)
    ED = (edb0, edb1)
    SR = (srcb0, srcb1)
    DS = (dstb0, dstb1)
    FI = (fidx0, fidx1)
    DI = (didx0, didx1)
    DC = (dsc0, dsc1)
    EX = (exb0, exb1)
    SID = (sid0, sid1)
    SG = (sg0, sg1)
    SS = (ss0, ss1)

    def _z2(i, _):
        for v in range(8):
            zbuf2[i, pl.ds(v * 16, 16)] = jnp.zeros((16,), jnp.float32)
        return _
    lax.fori_loop(0, ZR, _z2, None)

    def _z1(i, _):
        zbufn[pl.ds(i * 16, 16)] = jnp.zeros((16,), jnp.float32)
        return _
    lax.fori_loop(0, N // 16, _z1, None)

    for r in (0, 1):
        row_base = s * RPT
        for q in range(RPT // ZR):
            pltpu.async_copy(zbuf2, acc.at[pl.ds(row_base + q * ZR, ZR)], sg0)
        for q in range(RPT // ZR):
            pltpu.make_async_copy(
                zbuf2, acc.at[pl.ds(row_base + q * ZR, ZR)], sg0).wait()

        @pl.when(s == 15)
        def _ztail():
            pltpu.sync_copy(zbuf2.at[pl.ds(0, 16)], acc.at[pl.ds(N - 16, 16)])

        @pl.when(s == 0)
        def _zd():
            pltpu.sync_copy(zbufn, den0)

        @pl.when(s == 1)
        def _zd1():
            pltpu.sync_copy(zbufn, den1)

        plsc.subcore_barrier()

        feat_base = (2 * r + c) * N
        srcs_hbm = ei_hbm.at[r, 0]
        dsts_hbm = ei_hbm.at[r, 1]

        def issue_ids(k, p):
            base = s * EPT + k * CH
            pltpu.async_copy(srcs_hbm.at[pl.ds(base, CH)], SR[p], SID[p])
            pltpu.async_copy(dsts_hbm.at[pl.ds(base, CH)], DS[p], SID[p])

        def wait_ids(p):
            pltpu.make_async_copy(srcs_hbm.at[pl.ds(0, CH)], SR[p], SID[p]).wait()
            pltpu.make_async_copy(dsts_hbm.at[pl.ds(0, CH)], DS[p], SID[p]).wait()

        def build_and_gather(p):
            for g in range(CH // 16):
                sl = pl.ds(g * 16, 16)
                sv = SR[p][sl]
                dv = DS[p][sl]
                FI[p][sl] = sv + feat_base
                DI[p][sl] = dv + feat_base
                DC[p][sl] = dv
            pltpu.async_copy(feat_hbm.at[FI[p]], GB[p], SG[p])
            pltpu.async_copy(elr_hbm.at[FI[p]], ES[p], SG[p])
            pltpu.async_copy(elr_hbm.at[DI[p]], ED[p], SG[p])

        def wait_gather(p):
            pltpu.make_async_copy(feat_hbm.at[FI[p]], GB[p], SG[p]).wait()
            pltpu.make_async_copy(elr_hbm.at[FI[p]], ES[p], SG[p]).wait()
            pltpu.make_async_copy(elr_hbm.at[DI[p]], ED[p], SG[p]).wait()

        def process(p):
            def _grp(g, _):
                i16 = lax.iota(jnp.int32, 16) + g * 16
                exv = []
                for j in range(2):
                    el = plsc.load_gather(
                        ES[p], [i16, jnp.full((16,), j, jnp.int32)])
                    er = plsc.load_gather(
                        ED[p], [i16, jnp.full((16,), 2 + j, jnp.int32)])
                    e = el + er
                    e = jnp.where(e >= 0, e, e * NEG_SLOPE)
                    ex = jnp.exp(e)
                    EX[p][j, pl.ds(g * 16, 16)] = ex
                    exv.append(ex)
                for l in range(16):
                    i = g * 16 + l
                    w0 = exv[0][l]
                    w1 = exv[1][l]
                    for v in range(4):
                        GB[p][i, pl.ds(v * 16, 16)] = (
                            GB[p][i, pl.ds(v * 16, 16)] * w0)
                        GB[p][i, pl.ds(64 + v * 16, 16)] = (
                            GB[p][i, pl.ds(64 + v * 16, 16)] * w1)
                return _
            lax.fori_loop(0, CH // 16, _grp, None)

        def issue_scatter(p):
            pltpu.async_copy(GB[p], acc.at[DC[p]], SS[p], add=True)
            pltpu.async_copy(EX[p].at[0], den0.at[DC[p]], SS[p], add=True)
            pltpu.async_copy(EX[p].at[1], den1.at[DC[p]], SS[p], add=True)

        def drain_scatter(p):
            pltpu.make_async_copy(GB[p], acc.at[DC[p]], SS[p]).wait()
            pltpu.make_async_copy(EX[p].at[0], den0.at[DC[p]], SS[p]).wait()
            pltpu.make_async_copy(EX[p].at[1], den1.at[DC[p]], SS[p]).wait()

        issue_ids(0, 0)
        wait_ids(0)
        build_and_gather(0)
        issue_ids(1, 1)
        wait_ids(1)
        build_and_gather(1)
        wait_gather(0)
        process(0)
        issue_scatter(0)
        issue_ids(2, 0)

        def _main(m, _):
            k = 2 * m
            issue_ids(k + 3, 1)
            drain_scatter(0)
            wait_ids(0)
            build_and_gather(0)
            wait_gather(1)
            process(1)
            issue_scatter(1)
            issue_ids(k + 4, 0)
            drain_scatter(1)
            wait_ids(1)
            build_and_gather(1)
            wait_gather(0)
            process(0)
            issue_scatter(0)
            return _
        lax.fori_loop(0, (NCHUNK - 3) // 2, _main, None)

        drain_scatter(0)
        wait_ids(0)
        build_and_gather(0)
        wait_gather(1)
        process(1)
        issue_scatter(1)
        drain_scatter(1)
        wait_gather(0)
        process(0)
        issue_scatter(0)
        drain_scatter(0)

        plsc.subcore_barrier()

        r0 = s * RPT
        pltpu.sync_copy(acc.at[pl.ds(r0, RPT)],
                        out_hbm.at[r, pl.ds(c * N + r0, RPT)])

        @pl.when(s == 15)
        def _wtail():
            pltpu.sync_copy(acc.at[pl.ds(N - 16, 16)],
                            out_hbm.at[r, pl.ds(c * N + N - 16, 16)])

        @pl.when(s == 0)
        def _wd0():
            pltpu.sync_copy(den0, den_hbm.at[r, c, 0])

        @pl.when(s == 1)
        def _wd1():
            pltpu.sync_copy(den1, den_hbm.at[r, c, 1])

        plsc.subcore_barrier()


def _stage2(feat_cat, elr, ei_all):
    mesh = plsc.VectorSubcoreMesh(core_axis_name="c", subcore_axis_name="s")
    fn = pl.kernel(
        _sc_body,
        out_type=(jax.ShapeDtypeStruct((2, 2 * N, 128), jnp.float32),
                  jax.ShapeDtypeStruct((2, 2, 2, N), jnp.float32)),
        mesh=mesh,
        compiler_params=pltpu.CompilerParams(use_tc_tiling_on_sc=False,
                                             needs_layout_passes=False),
        scratch_types=(
            [pltpu.VMEM((CH, 128), jnp.float32)] * 2 +
            [pltpu.VMEM((CH, 16), jnp.float32)] * 4 +
            [pltpu.VMEM((CH,), jnp.int32)] * 10 +
            [pltpu.VMEM((2, CH), jnp.float32)] * 2 +
            [pltpu.VMEM((ZR, 128), jnp.float32),
             pltpu.VMEM((N,), jnp.float32),
             pltpu.VMEM_SHARED((N, 128), jnp.float32),
             pltpu.VMEM_SHARED((N,), jnp.float32),
             pltpu.VMEM_SHARED((N,), jnp.float32)] +
            [pltpu.SemaphoreType.DMA] * 6
        ),
    )
    return fn(feat_cat, elr, ei_all)


def _tc2_body(h_ref, den_ref, wm_ref, bm_ref, o_ref):
    hb = h_ref[...]
    dn = den_ref[...]
    dn = jnp.where(dn == 0.0, 1.0, dn)
    inv = 1.0 / dn
    parts = []
    for q in range(4):
        i0 = jnp.broadcast_to(inv[:, 2 * q:2 * q + 1], (BN, 64))
        i1 = jnp.broadcast_to(inv[:, 2 * q + 1:2 * q + 2], (BN, 64))
        parts.append(hb[q] * jnp.concatenate([i0, i1], axis=1))
    cat = jnp.concatenate(parts, axis=1)
    o_ref[...] = lax.dot_general(cat, wm_ref[...], (((1,), (1,)), ((), ())),
                                 preferred_element_type=jnp.float32) + bm_ref[...]


def _stage3(hcat, den8, Wm, bm):
    return pl.pallas_call(
        _tc2_body,
        grid=(N // BN,),
        in_specs=[
            pl.BlockSpec((4, BN, 128), lambda n: (0, n, 0)),
            pl.BlockSpec((BN, 8), lambda n: (n, 0)),
            pl.BlockSpec((F, 2 * H * F), lambda n: (0, 0)),
            pl.BlockSpec((1, F), lambda n: (0, 0)),
        ],
        out_specs=pl.BlockSpec((BN, F), lambda n: (n, 0)),
        out_shape=jax.ShapeDtypeStruct((N, F), jnp.float32),
    )(hcat, den8, Wm, bm)


def _build_b(attn_l, attn_r):
    z = jnp.zeros((64,), jnp.float32)
    zcol = jnp.zeros((128,), jnp.float32)
    per_c = []
    for c in range(2):
        cols = [jnp.concatenate([attn_l[0, 2 * c], z]),
                jnp.concatenate([z, attn_l[0, 2 * c + 1]]),
                jnp.concatenate([attn_r[0, 2 * c], z]),
                jnp.concatenate([z, attn_r[0, 2 * c + 1]])] + [zcol] * 12
        per_c.append(jnp.stack(cols, axis=1))
    return jnp.stack(per_c)


def kernel(x, edge_index_rel0, edge_index_rel1, W1, attn_l1, attn_r1,
           W2, attn_l2, attn_r2, Wm, bm):
    Wstack = jnp.stack([W1, W2])
    Ball = jnp.stack([_build_b(attn_l1, attn_r1),
                      _build_b(attn_l2, attn_r2)])
    ei_all = jnp.stack([edge_index_rel0, edge_index_rel1])

    feat, elr = _stage1(x, Wstack, Ball)
    feat_cat = feat.reshape(4 * N, 128)
    elr_cat = elr.reshape(4 * N, 16)

    out_raw, den_all = _stage2(feat_cat, elr_cat, ei_all)
    hcat = out_raw.reshape(4, N, 128)
    den8 = jnp.transpose(den_all.reshape(8, N), (1, 0))

    return _stage3(hcat, den8, Wm, bm.reshape(1, F))


# speedup vs baseline: 1.2148x; 1.0002x over previous
import jax
import jax.numpy as jnp
from jax import lax
from jax.experimental import pallas as pl
from jax.experimental.pallas import tpu as pltpu
from jax.experimental.pallas import tpu_sc as plsc

N = 10000
E = 160000
D = 256
H = 4
F = 64
NEG_SLOPE = 0.2

BN = 1000
CH = 80
EPT = E // 16
NCHUNK = EPT // CH
RPT = 624
ZR = 52


def _tc1_body(x_ref, w_ref, b_ref, feat_ref, elr_ref):
    xb = x_ref[...]
    wb = w_ref[0]
    fb = lax.dot_general(xb, wb, (((1,), (1,)), ((), ())),
                         preferred_element_type=jnp.float32)
    feat_ref[0, 0] = fb
    elr_ref[0, 0] = jnp.dot(fb, b_ref[0, 0], preferred_element_type=jnp.float32)


def _tc1_specs():
    return dict(
        in_specs=[
            pl.BlockSpec((BN, D), lambda n, r, c: (n, 0)),
            pl.BlockSpec((1, 128, D), lambda n, r, c: (r, c, 0)),
            pl.BlockSpec((1, 1, 128, 16), lambda n, r, c: (r, c, 0, 0)),
        ],
        out_specs=[
            pl.BlockSpec((1, 1, BN, 128), lambda n, r, c: (r, c, n, 0)),
            pl.BlockSpec((1, 1, BN, 16), lambda n, r, c: (r, c, n, 0)),
        ],
        out_shape=[
            jax.ShapeDtypeStruct((2, 2, N, 128), jnp.float32),
            jax.ShapeDtypeStruct((2, 2, N, 16), jnp.float32),
        ],
    )


def _stage1(x, Wstack, Ball):
    return pl.pallas_call(
        _tc1_body, grid=(N // BN, 2, 2), **_tc1_specs(),
    )(x, Wstack, Ball)


def _sc_body(feat_hbm, elr_hbm, ei_hbm, out_hbm, den_hbm,
             gbuf0, gbuf1, esb0, esb1, edb0, edb1,
             srcb0, srcb1, dstb0, dstb1, fidx0, fidx1, didx0, didx1,
             dsc0, dsc1, exb0, exb1, zbuf2, zbufn, acc, den0, den1,
             sid0, sid1, sg0, sg1, ss0, ss1):
    c = lax.axis_index("c")
    s = lax.axis_index("s")
    GB = (gbuf0, gbuf1)
    ES = (esb0, esb1)
    ED = (edb0, edb1)
    SR = (srcb0, srcb1)
    DS = (dstb0, dstb1)
    FI = (fidx0, fidx1)
    DI = (didx0, didx1)
    DC = (dsc0, dsc1)
    EX = (exb0, exb1)
    SID = (sid0, sid1)
    SG = (sg0, sg1)
    SS = (ss0, ss1)

    def _z2(i, _):
        for v in range(8):
            zbuf2[i, pl.ds(v * 16, 16)] = jnp.zeros((16,), jnp.float32)
        return _
    lax.fori_loop(0, ZR, _z2, None)

    def _z1(i, _):
        zbufn[pl.ds(i * 16, 16)] = jnp.zeros((16,), jnp.float32)
        return _
    lax.fori_loop(0, N // 16, _z1, None)

    for r in (0, 1):
        row_base = s * RPT
        for q in range(RPT // ZR):
            pltpu.async_copy(zbuf2, acc.at[pl.ds(row_base + q * ZR, ZR)], sg0)
        for q in range(RPT // ZR):
            pltpu.make_async_copy(
                zbuf2, acc.at[pl.ds(row_base + q * ZR, ZR)], sg0).wait()

        @pl.when(s == 15)
        def _ztail():
            pltpu.sync_copy(zbuf2.at[pl.ds(0, 16)], acc.at[pl.ds(N - 16, 16)])

        @pl.when(s == 0)
        def _zd():
            pltpu.sync_copy(zbufn, den0)

        @pl.when(s == 1)
        def _zd1():
            pltpu.sync_copy(zbufn, den1)

        plsc.subcore_barrier()

        feat_base = (2 * r + c) * N
        srcs_hbm = ei_hbm.at[r, 0]
        dsts_hbm = ei_hbm.at[r, 1]

        def issue_ids(k, p):
            base = s * EPT + k * CH
            pltpu.async_copy(srcs_hbm.at[pl.ds(base, CH)], SR[p], SID[p])
            pltpu.async_copy(dsts_hbm.at[pl.ds(base, CH)], DS[p], SID[p])

        def wait_ids(p):
            pltpu.make_async_copy(srcs_hbm.at[pl.ds(0, CH)], SR[p], SID[p]).wait()
            pltpu.make_async_copy(dsts_hbm.at[pl.ds(0, CH)], DS[p], SID[p]).wait()

        def build_and_gather(p):
            for g in range(CH // 16):
                sl = pl.ds(g * 16, 16)
                sv = SR[p][sl]
                dv = DS[p][sl]
                FI[p][sl] = sv + feat_base
                DI[p][sl] = dv + feat_base
                DC[p][sl] = dv
            pltpu.async_copy(feat_hbm.at[FI[p]], GB[p], SG[p])
            pltpu.async_copy(elr_hbm.at[FI[p]], ES[p], SG[p])
            pltpu.async_copy(elr_hbm.at[DI[p]], ED[p], SG[p])

        def wait_gather(p):
            pltpu.make_async_copy(feat_hbm.at[FI[p]], GB[p], SG[p]).wait()
            pltpu.make_async_copy(elr_hbm.at[FI[p]], ES[p], SG[p]).wait()
            pltpu.make_async_copy(elr_hbm.at[DI[p]], ED[p], SG[p]).wait()

        def process(p):
            def _grp(g, _):
                i16 = lax.iota(jnp.int32, 16) + g * 16
                exv = []
                for j in range(2):
                    el = plsc.load_gather(
                        ES[p], [i16, jnp.full((16,), j, jnp.int32)])
                    er = plsc.load_gather(
                        ED[p], [i16, jnp.full((16,), 2 + j, jnp.int32)])
                    e = el + er
                    e = jnp.where(e >= 0, e, e * NEG_SLOPE)
                    ex = jnp.exp(e)
                    EX[p][j, pl.ds(g * 16, 16)] = ex
                    exv.append(ex)
                for l in range(16):
                    i = g * 16 + l
                    w0 = exv[0][l]
                    w1 = exv[1][l]
                    for v in range(4):
                        GB[p][i, pl.ds(v * 16, 16)] = (
                            GB[p][i, pl.ds(v * 16, 16)] * w0)
                        GB[p][i, pl.ds(64 + v * 16, 16)] = (
                            GB[p][i, pl.ds(64 + v * 16, 16)] * w1)
                return _
            lax.fori_loop(0, CH // 16, _grp, None)

        def issue_scatter(p):
            pltpu.async_copy(GB[p], acc.at[DC[p]], SS[p], add=True)
            pltpu.async_copy(EX[p].at[0], den0.at[DC[p]], SS[p], add=True)
            pltpu.async_copy(EX[p].at[1], den1.at[DC[p]], SS[p], add=True)

        def drain_scatter(p):
            pltpu.make_async_copy(GB[p], acc.at[DC[p]], SS[p]).wait()
            pltpu.make_async_copy(EX[p].at[0], den0.at[DC[p]], SS[p]).wait()
            pltpu.make_async_copy(EX[p].at[1], den1.at[DC[p]], SS[p]).wait()

        issue_ids(0, 0)
        wait_ids(0)
        build_and_gather(0)
        issue_ids(1, 1)
        wait_ids(1)
        build_and_gather(1)
        wait_gather(0)
        process(0)
        issue_scatter(0)
        issue_ids(2, 0)

        def _main(m, _):
            k = 2 * m
            issue_ids(k + 3, 1)
            drain_scatter(0)
            wait_ids(0)
            build_and_gather(0)
            wait_gather(1)
            process(1)
            issue_scatter(1)
            issue_ids(k + 4, 0)
            drain_scatter(1)
            wait_ids(1)
            build_and_gather(1)
            wait_gather(0)
            process(0)
            issue_scatter(0)
            return _
        lax.fori_loop(0, (NCHUNK - 3) // 2, _main, None)

        drain_scatter(0)
        wait_ids(0)
        build_and_gather(0)
        wait_gather(1)
        process(1)
        issue_scatter(1)
        drain_scatter(1)
        wait_gather(0)
        process(0)
        issue_scatter(0)
        drain_scatter(0)

        plsc.subcore_barrier()

        r0 = s * RPT
        pltpu.sync_copy(acc.at[pl.ds(r0, RPT)],
                        out_hbm.at[r, pl.ds(c * N + r0, RPT)])

        @pl.when(s == 15)
        def _wtail():
            pltpu.sync_copy(acc.at[pl.ds(N - 16, 16)],
                            out_hbm.at[r, pl.ds(c * N + N - 16, 16)])

        @pl.when(s == 0)
        def _wd0():
            pltpu.sync_copy(den0, den_hbm.at[r, c, 0])

        @pl.when(s == 1)
        def _wd1():
            pltpu.sync_copy(den1, den_hbm.at[r, c, 1])

        plsc.subcore_barrier()


def _stage2(feat_cat, elr, ei_all):
    mesh = plsc.VectorSubcoreMesh(core_axis_name="c", subcore_axis_name="s")
    fn = pl.kernel(
        _sc_body,
        out_type=(jax.ShapeDtypeStruct((2, 2 * N, 128), jnp.float32),
                  jax.ShapeDtypeStruct((2, 2, 2, N), jnp.float32)),
        mesh=mesh,
        compiler_params=pltpu.CompilerParams(use_tc_tiling_on_sc=False,
                                             needs_layout_passes=False),
        scratch_types=(
            [pltpu.VMEM((CH, 128), jnp.float32)] * 2 +
            [pltpu.VMEM((CH, 16), jnp.float32)] * 4 +
            [pltpu.VMEM((CH,), jnp.int32)] * 10 +
            [pltpu.VMEM((2, CH), jnp.float32)] * 2 +
            [pltpu.VMEM((ZR, 128), jnp.float32),
             pltpu.VMEM((N,), jnp.float32),
             pltpu.VMEM_SHARED((N, 128), jnp.float32),
             pltpu.VMEM_SHARED((N,), jnp.float32),
             pltpu.VMEM_SHARED((N,), jnp.float32)] +
            [pltpu.SemaphoreType.DMA] * 6
        ),
    )
    return fn(feat_cat, elr, ei_all)


def _tc2_body(h_ref, den_ref, wm_ref, bm_ref, o_ref):
    hb = h_ref[...]
    dn = den_ref[...]
    dn = jnp.where(dn == 0.0, 1.0, dn)
    inv = 1.0 / dn
    parts = []
    for q in range(4):
        i0 = jnp.broadcast_to(inv[:, 2 * q:2 * q + 1], (BN, 64))
        i1 = jnp.broadcast_to(inv[:, 2 * q + 1:2 * q + 2], (BN, 64))
        parts.append(hb[q] * jnp.concatenate([i0, i1], axis=1))
    cat = jnp.concatenate(parts, axis=1)
    o_ref[...] = lax.dot_general(cat, wm_ref[...], (((1,), (1,)), ((), ())),
                                 preferred_element_type=jnp.float32) + bm_ref[...]


def _stage3(hcat, den8, Wm, bm):
    return pl.pallas_call(
        _tc2_body,
        grid=(N // BN,),
        in_specs=[
            pl.BlockSpec((4, BN, 128), lambda n: (0, n, 0)),
            pl.BlockSpec((BN, 8), lambda n: (n, 0)),
            pl.BlockSpec((F, 2 * H * F), lambda n: (0, 0)),
            pl.BlockSpec((1, F), lambda n: (0, 0)),
        ],
        out_specs=pl.BlockSpec((BN, F), lambda n: (n, 0)),
        out_shape=jax.ShapeDtypeStruct((N, F), jnp.float32),
    )(hcat, den8, Wm, bm)


def _build_b(attn_l, attn_r):
    z = jnp.zeros((64,), jnp.float32)
    zcol = jnp.zeros((128,), jnp.float32)
    per_c = []
    for c in range(2):
        cols = [jnp.concatenate([attn_l[0, 2 * c], z]),
                jnp.concatenate([z, attn_l[0, 2 * c + 1]]),
                jnp.concatenate([attn_r[0, 2 * c], z]),
                jnp.concatenate([z, attn_r[0, 2 * c + 1]])] + [zcol] * 12
        per_c.append(jnp.stack(cols, axis=1))
    return jnp.stack(per_c)


def kernel(x, edge_index_rel0, edge_index_rel1, W1, attn_l1, attn_r1,
           W2, attn_l2, attn_r2, Wm, bm):
    Wstack = jnp.stack([W1, W2])
    Ball = jnp.stack([_build_b(attn_l1, attn_r1),
                      _build_b(attn_l2, attn_r2)])
    ei_all = jnp.stack([edge_index_rel0, edge_index_rel1])

    feat, elr = _stage1(x, Wstack, Ball)
    feat_cat = feat.reshape(4 * N, 128)
    elr_cat = elr.reshape(4 * N, 16)

    out_raw, den_all = _stage2(feat_cat, elr_cat, ei_all)
    hcat = out_raw.reshape(4, N, 128)
    den8 = jnp.transpose(den_all.reshape(8, N), (1, 0))

    return _stage3(hcat, den8, Wm, bm.reshape(1, F))
